# baseline (speedup 1.0000x reference)
import functools

import jax
import jax.numpy as jnp
from jax.experimental import pallas as pl
from jax.experimental.pallas import tpu as pltpu

B, C, L = 16, 8, 65536
ROWS = 8
PAD = 128
TREND_K = 11
LOCAL_K = 5
TREND_H = 5
LOCAL_H = 2
TREND_SCALING = 0.6
DETAIL = 0.85
SPIKE_T = 3.5
SPIKE_D = 0.35
EPS = 1e-6
ITERS = 2


def _edge_matrices(h, k):
    i = jax.lax.broadcasted_iota(jnp.int32, (256, 128), 0)
    p = jax.lax.broadcasted_iota(jnp.int32, (256, 128), 1)
    inv_k = 1.0 / float(k)
    left = ((jnp.abs(i - p) <= h).astype(jnp.float32)
            + ((i >= 1) & (i <= h - p)).astype(jnp.float32)) * inv_k
    q = 128 + p
    right = ((jnp.abs(i - q) <= h).astype(jnp.float32)
             + ((i >= 510 - q - h) & (i <= 254)).astype(jnp.float32)) * inv_k
    return left, right


def _denoise_body(x_ref, o_ref, ps_ref):
    t_left, t_right = _edge_matrices(TREND_H, TREND_K)
    l_left, l_right = _edge_matrices(LOCAL_H, LOCAL_K)
    zeros = jnp.zeros((ROWS, PAD), dtype=jnp.float32)

    def one_iter(v):
        ps_ref[:, 0:PAD] = zeros
        ps_ref[:, PAD:PAD + L] = v
        ps_ref[:, PAD + L:PAD + L + PAD] = zeros

        s2 = ps_ref[:, PAD - 2:PAD - 2 + L]
        for j in (-1, 0, 1, 2):
            s2 = s2 + ps_ref[:, PAD + j:PAD + j + L]
        s5 = s2
        for j in (-5, -4, -3, 3, 4, 5):
            s5 = s5 + ps_ref[:, PAD + j:PAD + j + L]
        local = s2 * (1.0 / LOCAL_K)
        trend = s5 * (1.0 / TREND_K)

        xl = v[:, :256]
        xr = v[:, L - 256:]
        dot = functools.partial(
            jax.lax.dot_general,
            dimension_numbers=(((1,), (0,)), ((), ())),
            preferred_element_type=jnp.float32,
            precision=jax.lax.Precision.HIGHEST)
        local = jnp.concatenate(
            [dot(xl, l_left), local[:, 128:L - 128], dot(xr, l_right)], axis=1)
        trend = jnp.concatenate(
            [dot(xl, t_left), trend[:, 128:L - 128], dot(xr, t_right)], axis=1)

        r = v - local
        mean = jnp.sum(r, axis=1, keepdims=True) * (1.0 / L)
        var = jnp.sum((r - mean) ** 2, axis=1, keepdims=True) * (1.0 / (L - 1))
        scale = jnp.maximum(jnp.sqrt(var), EPS)
        thr = scale * SPIKE_T
        r = jnp.where(jnp.abs(r) > thr, r * SPIKE_D, r) * DETAIL
        return (1.0 - TREND_SCALING) * local + TREND_SCALING * trend + r

    v = x_ref[...]
    for _ in range(ITERS):
        v = one_iter(v)
    o_ref[...] = v


@jax.jit
def kernel(x):
    xf = x.astype(jnp.float32).reshape(B * C, L)
    out = pl.pallas_call(
        _denoise_body,
        grid=(B * C // ROWS,),
        in_specs=[pl.BlockSpec((ROWS, L), lambda i: (i, 0))],
        out_specs=pl.BlockSpec((ROWS, L), lambda i: (i, 0)),
        out_shape=jax.ShapeDtypeStruct((B * C, L), jnp.float32),
        scratch_shapes=[pltpu.VMEM((ROWS, L + 2 * PAD), jnp.float32)],
        compiler_params=pltpu.CompilerParams(
            dimension_semantics=("parallel",)),
    )(xf)
    return out.reshape(B, C, L)


# speedup vs baseline: 1.2157x; 1.2157x over previous
import functools

import jax
import jax.numpy as jnp
from jax.experimental import pallas as pl
from jax.experimental.pallas import tpu as pltpu

B, C, L = 16, 8, 65536
ROWS = 8
PAD = 128
TREND_K = 11
LOCAL_K = 5
TREND_H = 5
LOCAL_H = 2
TREND_SCALING = 0.6
DETAIL = 0.85
SPIKE_T = 3.5
SPIKE_D = 0.35
EPS = 1e-6
ITERS = 2


def _edge_matrices(h, k):
    i = jax.lax.broadcasted_iota(jnp.int32, (256, 128), 0)
    p = jax.lax.broadcasted_iota(jnp.int32, (256, 128), 1)
    inv_k = 1.0 / float(k)
    left = ((jnp.abs(i - p) <= h).astype(jnp.float32)
            + ((i >= 1) & (i <= h - p)).astype(jnp.float32)) * inv_k
    q = 128 + p
    right = ((jnp.abs(i - q) <= h).astype(jnp.float32)
             + ((i >= 510 - q - h) & (i <= 254)).astype(jnp.float32)) * inv_k
    return left, right


def _denoise_body(x_ref, o_ref, ps_ref):
    t_left, t_right = _edge_matrices(TREND_H, TREND_K)
    l_left, l_right = _edge_matrices(LOCAL_H, LOCAL_K)
    zeros = jnp.zeros((ROWS, PAD), dtype=jnp.float32)

    def one_iter(v):
        ps_ref[:, 0:PAD] = zeros
        ps_ref[:, PAD:PAD + L] = v
        ps_ref[:, PAD + L:PAD + L + PAD] = zeros

        t = {j: ps_ref[:, PAD + j:PAD + j + L] for j in range(-5, 6)}
        s2 = ((t[-2] + t[-1]) + (t[0] + t[1])) + t[2]
        s5 = s2 + (((t[-5] + t[-4]) + (t[-3] + t[3])) + (t[4] + t[5]))
        local = s2 * (1.0 / LOCAL_K)
        trend = s5 * (1.0 / TREND_K)

        xl = v[:, :256]
        xr = v[:, L - 256:]
        dot = functools.partial(
            jax.lax.dot_general,
            dimension_numbers=(((1,), (0,)), ((), ())),
            preferred_element_type=jnp.float32,
            precision=jax.lax.Precision.HIGHEST)
        local = jnp.concatenate(
            [dot(xl, l_left), local[:, 128:L - 128], dot(xr, l_right)], axis=1)
        trend = jnp.concatenate(
            [dot(xl, t_left), trend[:, 128:L - 128], dot(xr, t_right)], axis=1)

        r = v - local
        mean = jnp.sum(r, axis=1, keepdims=True) * (1.0 / L)
        var = jnp.sum((r - mean) ** 2, axis=1, keepdims=True) * (1.0 / (L - 1))
        scale = jnp.maximum(jnp.sqrt(var), EPS)
        thr = scale * SPIKE_T
        r = jnp.where(jnp.abs(r) > thr, r * SPIKE_D, r) * DETAIL
        return (1.0 - TREND_SCALING) * local + TREND_SCALING * trend + r

    v = x_ref[...]
    for _ in range(ITERS):
        v = one_iter(v)
    o_ref[...] = v


@jax.jit
def kernel(x):
    xf = x.astype(jnp.float32).reshape(B * C, L)
    out = pl.pallas_call(
        _denoise_body,
        grid=(B * C // ROWS,),
        in_specs=[pl.BlockSpec((ROWS, L), lambda i: (i, 0))],
        out_specs=pl.BlockSpec((ROWS, L), lambda i: (i, 0)),
        out_shape=jax.ShapeDtypeStruct((B * C, L), jnp.float32),
        scratch_shapes=[pltpu.VMEM((ROWS, L + 2 * PAD), jnp.float32)],
        compiler_params=pltpu.CompilerParams(
            dimension_semantics=("parallel",)),
    )(xf)
    return out.reshape(B, C, L)
